# Initial kernel scaffold; baseline (speedup 1.0000x reference)
import jax
import jax.numpy as jnp
from jax.experimental import pallas as pl


def kernel(cls_feats, label_feats, hiddens, audio_embedding, image_results, W_ap, b_ap, Wq, bq, Wk, bk, Wv, bv, Wo, bo, ln_g, ln_b, W1, b1, W2, b2, Wfc, bfc, eW1, eb1, eW2, eb2, Wmg, bmg, Wg1, bg1, Wg2, bg2):
    raise NotImplementedError("write your pallas kernel here")


# speedup vs baseline: 4.9313x; 4.9313x over previous
import functools

import jax
import jax.numpy as jnp
from jax.experimental import pallas as pl

_B = 4096
_D = 768
_AD = 128
_H = 512
_NC = 5
_E = 4
_MH = 128
_GH = 128
_EH = _E * _MH


def _dot(a, b):
    return jnp.dot(a, b, preferred_element_type=jnp.float32)


def _prep_kernel(W_ap, Wv, Wo, Wg1b, bg1, b_ap, bv, bo, Wmg, bmg, eW1c,
                 eb1c, eb2,
                 WapWg1b_o, bg1t_o, Wmg128_o, cq_o, cl_o, We1_o, ch1_o,
                 ch2_o, eb2s_o):
    wap = W_ap[...]
    WapWg1b_o[...] = _dot(wap, Wg1b[...])
    bg1t_o[...] = bg1[...] + _dot(b_ap[...], Wg1b[...])
    wav = _dot(wap, Wv[...])
    wapvo = _dot(wav, Wo[...])
    bavo = _dot(_dot(b_ap[...], Wv[...]), Wo[...])
    bvo = _dot(bv[...], Wo[...]) + bo[...]
    Wmg128_o[...] = _dot(wapvo, Wmg[...])
    cq_o[...] = _dot(bavo, Wmg[...])
    cl_o[...] = _dot(bvo, Wmg[...]) + bmg[...]
    We1_o[...] = _dot(wapvo, eW1c[...])
    ch1_o[...] = _dot(bavo, eW1c[...])
    ch2_o[...] = _dot(bvo, eW1c[...]) + eb1c[...]
    eb2s_o[...] = jnp.sum(eb2[...], axis=0, keepdims=True)


def _layer_norm(x, g, b):
    mu = jnp.mean(x, axis=-1, keepdims=True)
    xc = x - mu
    var = jnp.mean(xc * xc, axis=-1, keepdims=True)
    return g * (xc * jax.lax.rsqrt(var + 1e-5)) + b


def _main_kernel(lf, cif, a, xt, noise,
                 Wg1a, Wg1c, WapWg1b, bg1t, Wg2, bg2,
                 Wmg128, cq, cl, We1, ch1, ch2, eW2c, eb2s,
                 W1, b1, W2, b2, ln_g, ln_b, Wfc, bfc,
                 out):
    cif_v = cif[...]
    xi = jnp.zeros((cif_v.shape[0], _D), jnp.float32)
    for j in range(5):
        xi = xi + jnp.where(cif_v == j, lf[:, j * _D:(j + 1) * _D], 0.0)

    av = a[...]
    xtv = xt[...]
    g = (_dot(xi, Wg1a[...]) + _dot(av, WapWg1b[...]) + _dot(xtv, Wg1c[...])
         + bg1t[...])
    gl = _dot(jnp.maximum(g, 0.0), Wg2[...]) + bg2[...]
    gl = gl - jnp.max(gl, axis=-1, keepdims=True)
    ge = jnp.exp(gl)
    gw = ge / jnp.sum(ge, axis=-1, keepdims=True)
    gw1 = gw[:, 1:2]
    gw2 = gw[:, 2:3]

    lg = gw1 * (_dot(av, Wmg128[...]) + cq[...]) + cl[...] + noise[...]
    n0 = lg[:, 0:1]
    rest = lg[:, 1:4]
    m = jnp.max(rest, axis=-1, keepdims=True)
    r = jnp.sum((rest > n0).astype(jnp.float32), axis=-1, keepdims=True)
    sparse0 = jnp.where(r <= 1.0, jax.nn.sigmoid(n0 - m), 0.0)

    hm = jnp.maximum(gw1 * (_dot(av, We1[...]) + ch1[...]) + ch2[...], 0.0)
    y0 = sparse0 * (_dot(hm, eW2c[...]) + eb2s[...])

    lng = ln_g[...]
    lnb = ln_b[...]
    w1 = W1[...]
    b1v = b1[...]
    w2 = W2[...]
    b2v = b2[...]
    x = gw2 * xtv
    for _ in range(4):
        x = _layer_norm(y0 + x, lng, lnb)
        t = _dot(jnp.maximum(_dot(x, w1) + b1v, 0.0), w2) + b2v
        x = _layer_norm(t + x, lng, lnb)

    lf5 = _dot(x, Wfc[...]) + bfc[...]
    lf5 = lf5 - jnp.max(lf5, axis=-1, keepdims=True)
    e5 = jnp.exp(lf5)
    out[...] = e5 / jnp.sum(e5, axis=-1, keepdims=True)


@jax.jit
def kernel(cls_feats, label_feats, hiddens, audio_embedding, image_results,
           W_ap, b_ap, Wq, bq, Wk, bk, Wv, bv, Wo, bo, ln_g, ln_b,
           W1, b1, W2, b2, Wfc, bfc, eW1, eb1, eW2, eb2, Wmg, bmg,
           Wg1, bg1, Wg2, bg2):
    B = cls_feats.shape[0]
    f32 = jnp.float32

    noise = jax.random.normal(jax.random.key(1), (B, 1, _E), dtype=f32)
    noise = noise[:, 0, :] * 0.1

    eW1c = jnp.transpose(eW1, (1, 0, 2)).reshape(_D, _EH)
    eb1c = eb1.reshape(1, _EH)
    eW2c = eW2.reshape(_EH, _D)
    row = lambda v: v.reshape(1, -1)

    prep_out = pl.pallas_call(
        _prep_kernel,
        out_shape=(
            jax.ShapeDtypeStruct((_AD, _GH), f32),
            jax.ShapeDtypeStruct((1, _GH), f32),
            jax.ShapeDtypeStruct((_AD, _E), f32),
            jax.ShapeDtypeStruct((1, _E), f32),
            jax.ShapeDtypeStruct((1, _E), f32),
            jax.ShapeDtypeStruct((_AD, _EH), f32),
            jax.ShapeDtypeStruct((1, _EH), f32),
            jax.ShapeDtypeStruct((1, _EH), f32),
            jax.ShapeDtypeStruct((1, _D), f32),
        ),
    )(W_ap, Wv, Wo, Wg1[_D:2 * _D, :], row(bg1), row(b_ap), row(bv),
      row(bo), Wmg, row(bmg), eW1c, eb1c, eb2)
    (WapWg1b, bg1t, Wmg128, cq, cl, We1, ch1, ch2, eb2s) = prep_out

    R = 512
    grid = (B // R,)
    bspec = lambda shape: pl.BlockSpec(shape, lambda i: (0, 0))
    rspec = lambda w: pl.BlockSpec((R, w), lambda i: (i, 0))

    out = pl.pallas_call(
        _main_kernel,
        grid=grid,
        in_specs=[
            rspec(6 * _D),
            rspec(1),
            rspec(_AD),
            rspec(_D),
            rspec(_E),
            bspec((_D, _GH)),
            bspec((_D, _GH)),
            bspec((_AD, _GH)),
            bspec((1, _GH)),
            bspec((_GH, 3)),
            bspec((1, 3)),
            bspec((_AD, _E)),
            bspec((1, _E)),
            bspec((1, _E)),
            bspec((_AD, _EH)),
            bspec((1, _EH)),
            bspec((1, _EH)),
            bspec((_EH, _D)),
            bspec((1, _D)),
            bspec((_D, _H)),
            bspec((1, _H)),
            bspec((_H, _D)),
            bspec((1, _D)),
            bspec((1, _D)),
            bspec((1, _D)),
            bspec((_D, _NC)),
            bspec((1, _NC)),
        ],
        out_specs=rspec(_NC),
        out_shape=jax.ShapeDtypeStruct((B, _NC), f32),
    )(label_feats.reshape(B, 6 * _D),
      image_results.astype(f32).reshape(B, 1),
      audio_embedding[:, 0, :],
      hiddens[:, 0, :],
      noise,
      Wg1[:_D, :], Wg1[2 * _D:, :], WapWg1b, bg1t, Wg2, row(bg2),
      Wmg128, cq, cl, We1, ch1, ch2, eW2c, eb2s,
      W1, row(b1), W2, row(b2), row(ln_g), row(ln_b), Wfc, row(bfc))
    return out


# speedup vs baseline: 5.4011x; 1.0953x over previous
import functools

import jax
import jax.numpy as jnp
from jax.experimental import pallas as pl

_B = 4096
_D = 768
_AD = 128
_H = 512
_NC = 5
_E = 4
_MH = 128
_GH = 128
_EH = _E * _MH


def _dot(a, b):
    return jnp.dot(a, b, preferred_element_type=jnp.float32)


def _prep_kernel(W_ap, Wv, Wo, Wg1b, bg1, b_ap, bv, bo, Wmg, bmg, eW1c,
                 eb1c, eb2,
                 WapWg1b_o, bg1t_o, Wmg128_o, cq_o, cl_o, We1_o, ch1_o,
                 ch2_o, eb2s_o):
    wap = W_ap[...]
    WapWg1b_o[...] = _dot(wap, Wg1b[...])
    bg1t_o[...] = bg1[...] + _dot(b_ap[...], Wg1b[...])
    wav = _dot(wap, Wv[...])
    wapvo = _dot(wav, Wo[...])
    bavo = _dot(_dot(b_ap[...], Wv[...]), Wo[...])
    bvo = _dot(bv[...], Wo[...]) + bo[...]
    Wmg128_o[...] = _dot(wapvo, Wmg[...])
    cq_o[...] = _dot(bavo, Wmg[...])
    cl_o[...] = _dot(bvo, Wmg[...]) + bmg[...]
    We1_o[...] = _dot(wapvo, eW1c[...])
    ch1_o[...] = _dot(bavo, eW1c[...])
    ch2_o[...] = _dot(bvo, eW1c[...]) + eb1c[...]
    eb2s_o[...] = jnp.sum(eb2[...], axis=0, keepdims=True)


def _layer_norm(x, g, b):
    mu = jnp.mean(x, axis=-1, keepdims=True)
    xc = x - mu
    var = jnp.mean(xc * xc, axis=-1, keepdims=True)
    return g * (xc * jax.lax.rsqrt(var + 1e-5)) + b


def _main_kernel(lf, cif, a, xt, noise,
                 Wg1a, Wg1c, WapWg1b, bg1t, Wg2, bg2,
                 Wmg128, cq, cl, We1, ch1, ch2, eW2c, eb2s,
                 W1, b1, W2, b2, ln_g, ln_b, Wfc, bfc,
                 out):
    cif_v = cif[...]
    xi = jnp.zeros((cif_v.shape[0], _D), jnp.float32)
    for j in range(5):
        xi = xi + jnp.where(cif_v == j, lf[:, j, :], 0.0)

    av = a[:, 0, :]
    xtv = xt[:, 0, :]
    g = (_dot(xi, Wg1a[...]) + _dot(av, WapWg1b[...]) + _dot(xtv, Wg1c[...])
         + bg1t[...])
    gl = _dot(jnp.maximum(g, 0.0), Wg2[...]) + bg2[...]
    gl = gl - jnp.max(gl, axis=-1, keepdims=True)
    ge = jnp.exp(gl)
    gw = ge / jnp.sum(ge, axis=-1, keepdims=True)
    gw1 = gw[:, 1:2]
    gw2 = gw[:, 2:3]

    lg = gw1 * (_dot(av, Wmg128[...]) + cq[...]) + cl[...] + noise[...]
    n0 = lg[:, 0:1]
    rest = lg[:, 1:4]
    m = jnp.max(rest, axis=-1, keepdims=True)
    r = jnp.sum((rest > n0).astype(jnp.float32), axis=-1, keepdims=True)
    sparse0 = jnp.where(r <= 1.0, jax.nn.sigmoid(n0 - m), 0.0)

    hm = jnp.maximum(gw1 * (_dot(av, We1[...]) + ch1[...]) + ch2[...], 0.0)
    y0 = sparse0 * (_dot(hm, eW2c[...]) + eb2s[...])

    lng = ln_g[...]
    lnb = ln_b[...]
    w1 = W1[...]
    b1v = b1[...]
    w2 = W2[...]
    b2v = b2[...]
    x = gw2 * xtv
    for _ in range(4):
        x = _layer_norm(y0 + x, lng, lnb)
        t = _dot(jnp.maximum(_dot(x, w1) + b1v, 0.0), w2) + b2v
        x = _layer_norm(t + x, lng, lnb)

    lf5 = _dot(x, Wfc[...]) + bfc[...]
    lf5 = lf5 - jnp.max(lf5, axis=-1, keepdims=True)
    e5 = jnp.exp(lf5)
    out[...] = e5 / jnp.sum(e5, axis=-1, keepdims=True)


@jax.jit
def kernel(cls_feats, label_feats, hiddens, audio_embedding, image_results,
           W_ap, b_ap, Wq, bq, Wk, bk, Wv, bv, Wo, bo, ln_g, ln_b,
           W1, b1, W2, b2, Wfc, bfc, eW1, eb1, eW2, eb2, Wmg, bmg,
           Wg1, bg1, Wg2, bg2):
    B = cls_feats.shape[0]
    f32 = jnp.float32

    noise = jax.random.normal(jax.random.key(1), (B, 1, _E), dtype=f32)
    noise = noise[:, 0, :] * 0.1

    eW1c = jnp.transpose(eW1, (1, 0, 2)).reshape(_D, _EH)
    eb1c = eb1.reshape(1, _EH)
    eW2c = eW2.reshape(_EH, _D)
    row = lambda v: v.reshape(1, -1)

    prep_out = pl.pallas_call(
        _prep_kernel,
        out_shape=(
            jax.ShapeDtypeStruct((_AD, _GH), f32),
            jax.ShapeDtypeStruct((1, _GH), f32),
            jax.ShapeDtypeStruct((_AD, _E), f32),
            jax.ShapeDtypeStruct((1, _E), f32),
            jax.ShapeDtypeStruct((1, _E), f32),
            jax.ShapeDtypeStruct((_AD, _EH), f32),
            jax.ShapeDtypeStruct((1, _EH), f32),
            jax.ShapeDtypeStruct((1, _EH), f32),
            jax.ShapeDtypeStruct((1, _D), f32),
        ),
    )(W_ap, Wv, Wo, Wg1[_D:2 * _D, :], row(bg1), row(b_ap), row(bv),
      row(bo), Wmg, row(bmg), eW1c, eb1c, eb2)
    (WapWg1b, bg1t, Wmg128, cq, cl, We1, ch1, ch2, eb2s) = prep_out

    R = 512
    grid = (B // R,)
    bspec = lambda shape: pl.BlockSpec(shape, lambda i: (0, 0))
    rspec = lambda w: pl.BlockSpec((R, w), lambda i: (i, 0))

    out = pl.pallas_call(
        _main_kernel,
        grid=grid,
        in_specs=[
            pl.BlockSpec((R, 6, _D), lambda i: (i, 0, 0)),
            rspec(1),
            pl.BlockSpec((R, 1, _AD), lambda i: (i, 0, 0)),
            pl.BlockSpec((R, 1, _D), lambda i: (i, 0, 0)),
            rspec(_E),
            bspec((_D, _GH)),
            bspec((_D, _GH)),
            bspec((_AD, _GH)),
            bspec((1, _GH)),
            bspec((_GH, 3)),
            bspec((1, 3)),
            bspec((_AD, _E)),
            bspec((1, _E)),
            bspec((1, _E)),
            bspec((_AD, _EH)),
            bspec((1, _EH)),
            bspec((1, _EH)),
            bspec((_EH, _D)),
            bspec((1, _D)),
            bspec((_D, _H)),
            bspec((1, _H)),
            bspec((_H, _D)),
            bspec((1, _D)),
            bspec((1, _D)),
            bspec((1, _D)),
            bspec((_D, _NC)),
            bspec((1, _NC)),
        ],
        out_specs=rspec(_NC),
        out_shape=jax.ShapeDtypeStruct((B, _NC), f32),
    )(label_feats,
      image_results.astype(f32).reshape(B, 1),
      audio_embedding,
      hiddens,
      noise,
      Wg1[:_D, :], Wg1[2 * _D:, :], WapWg1b, bg1t, Wg2, row(bg2),
      Wmg128, cq, cl, We1, ch1, ch2, eW2c, eb2s,
      W1, row(b1), W2, row(b2), row(ln_g), row(ln_b), Wfc, row(bfc))
    return out
